# baseline (speedup 1.0000x reference)
import functools

import jax
import jax.numpy as jnp
from jax import lax
from jax.experimental import pallas as pl
from jax.experimental.pallas import tpu as pltpu
from jax.experimental.pallas import tpu_sc as plsc

N = 100000
HIDDEN = 128
NUM_CORES = 2
NUM_SUBCORES = 16
NW = NUM_CORES * NUM_SUBCORES
BLK = 512
GCHUNK = 128
NBLK = (N + BLK - 1) // BLK
BLOCKS_PER_W = (NBLK + NW - 1) // NW


def _sc_lookup(idx, table):
  mesh = plsc.VectorSubcoreMesh(core_axis_name="c", subcore_axis_name="s")

  @functools.partial(
      pl.kernel,
      mesh=mesh,
      out_type=jax.ShapeDtypeStruct((N, HIDDEN), jnp.float32),
      scratch_types=[
          pltpu.VMEM((BLK,), jnp.int32),
          pltpu.VMEM((BLK, HIDDEN), jnp.float32),
          pltpu.SemaphoreType.DMA,
      ],
  )
  def k(idx_hbm, table_hbm, out_hbm, idx_v, rows_v, sem):
    wid = lax.axis_index("s") * NUM_CORES + lax.axis_index("c")

    def body(step, carry):
      b = wid + step * NW

      @pl.when(b < NBLK)
      def _():
        base = jnp.minimum(b * BLK, N - BLK)
        pltpu.sync_copy(idx_hbm.at[pl.ds(base, BLK)], idx_v)
        for j in range(BLK // GCHUNK):
          pltpu.async_copy(
              table_hbm.at[idx_v.at[pl.ds(j * GCHUNK, GCHUNK)]],
              rows_v.at[pl.ds(j * GCHUNK, GCHUNK)],
              sem,
          ).wait()
        pltpu.sync_copy(rows_v, out_hbm.at[pl.ds(base, BLK)])

      return carry

    lax.fori_loop(0, BLOCKS_PER_W, body, 0)

  return k(idx, table)


def kernel(node_degree, degree_emb):
  return _sc_lookup(node_degree.astype(jnp.int32), degree_emb)


# speedup vs baseline: 1.0076x; 1.0076x over previous
import functools

import jax
import jax.numpy as jnp
from jax import lax
from jax.experimental import pallas as pl
from jax.experimental.pallas import tpu as pltpu
from jax.experimental.pallas import tpu_sc as plsc

N = 100000
HIDDEN = 128
NUM_CORES = 2
NUM_SUBCORES = 16
NW = NUM_CORES * NUM_SUBCORES
BLK = 392
STEPS = 8
GCHUNKS = (128, 128, 128, 8)


def _sc_lookup(idx, table):
  mesh = plsc.VectorSubcoreMesh(core_axis_name="c", subcore_axis_name="s")

  @functools.partial(
      pl.kernel,
      mesh=mesh,
      out_type=jax.ShapeDtypeStruct((N, HIDDEN), jnp.float32),
      scratch_types=[
          pltpu.VMEM((BLK,), jnp.int32),
          pltpu.VMEM((BLK,), jnp.int32),
          pltpu.VMEM((2, BLK, HIDDEN), jnp.float32),
          pltpu.SemaphoreType.DMA,
          pltpu.SemaphoreType.DMA,
          pltpu.SemaphoreType.DMA,
          pltpu.SemaphoreType.DMA,
          pltpu.SemaphoreType.DMA,
          pltpu.SemaphoreType.DMA,
      ],
  )
  def k(idx_hbm, table_hbm, out_hbm, idx_v0, idx_v1, rows_v,
        isem0, isem1, gsem0, gsem1, ssem0, ssem1):
    wid = lax.axis_index("s") * NUM_CORES + lax.axis_index("c")
    idx_bufs = (idx_v0, idx_v1)
    isems = (isem0, isem1)
    gsems = (gsem0, gsem1)
    ssems = (ssem0, ssem1)

    def base_of(step):
      return jnp.minimum((wid + step * NW) * BLK, N - BLK)

    idx_h = [None, None]
    idx_h[0] = pltpu.async_copy(
        idx_hbm.at[pl.ds(base_of(0), BLK)], idx_bufs[0], isems[0])

    store_h = [None, None]
    for step in range(STEPS):
      buf = step % 2
      nbuf = 1 - buf
      base = base_of(step)
      if step + 1 < STEPS:
        idx_h[nbuf] = pltpu.async_copy(
            idx_hbm.at[pl.ds(base_of(step + 1), BLK)],
            idx_bufs[nbuf], isems[nbuf])
      idx_h[buf].wait()
      if store_h[buf] is not None:
        store_h[buf].wait()
      ghs = []
      off = 0
      for g in GCHUNKS:
        ghs.append(pltpu.async_copy(
            table_hbm.at[idx_bufs[buf].at[pl.ds(off, g)]],
            rows_v.at[buf, pl.ds(off, g)],
            gsems[buf]))
        off += g
      for h in ghs:
        h.wait()
      store_h[buf] = pltpu.async_copy(
          rows_v.at[buf], out_hbm.at[pl.ds(base, BLK)], ssems[buf])

    store_h[0].wait()
    store_h[1].wait()

  return k(idx, table)


def kernel(node_degree, degree_emb):
  return _sc_lookup(node_degree.astype(jnp.int32), degree_emb)


# speedup vs baseline: 7.0095x; 6.9570x over previous
import functools

import jax
import jax.numpy as jnp
from jax import lax
from jax.experimental import pallas as pl
from jax.experimental.pallas import tpu as pltpu
from jax.experimental.pallas import tpu_sc as plsc

N = 100000
HIDDEN = 128
NUM_CORES = 2
NUM_SUBCORES = 16
NW = NUM_CORES * NUM_SUBCORES
BLK = 392
STEPS = 8
GCHUNKS = (128, 128, 128, 8)


def _sc_lookup(idx, table):
  mesh = plsc.VectorSubcoreMesh(core_axis_name="c", subcore_axis_name="s")

  @functools.partial(
      pl.kernel,
      mesh=mesh,
      out_type=jax.ShapeDtypeStruct((N, HIDDEN), jnp.float32),
      scratch_types=[
          pltpu.VMEM_SHARED((20, HIDDEN), jnp.float32),
          pltpu.VMEM((BLK,), jnp.int32),
          pltpu.VMEM((BLK,), jnp.int32),
          pltpu.VMEM((2, BLK, HIDDEN), jnp.float32),
          pltpu.SemaphoreType.DMA,
          pltpu.SemaphoreType.DMA,
          pltpu.SemaphoreType.DMA,
          pltpu.SemaphoreType.DMA,
          pltpu.SemaphoreType.DMA,
          pltpu.SemaphoreType.DMA,
      ],
  )
  def k(idx_hbm, table_hbm, out_hbm, table_v, idx_v0, idx_v1, rows_v,
        isem0, isem1, gsem0, gsem1, ssem0, ssem1):
    wid = lax.axis_index("s") * NUM_CORES + lax.axis_index("c")
    @pl.when(lax.axis_index("s") == 0)
    def _():
      pltpu.sync_copy(table_hbm, table_v)

    plsc.subcore_barrier()
    idx_bufs = (idx_v0, idx_v1)
    isems = (isem0, isem1)
    gsems = (gsem0, gsem1)
    ssems = (ssem0, ssem1)

    def base_of(step):
      return jnp.minimum((wid + step * NW) * BLK, N - BLK)

    idx_h = [None, None]
    idx_h[0] = pltpu.async_copy(
        idx_hbm.at[pl.ds(base_of(0), BLK)], idx_bufs[0], isems[0])

    store_h = [None, None]
    for step in range(STEPS):
      buf = step % 2
      nbuf = 1 - buf
      base = base_of(step)
      if step + 1 < STEPS:
        idx_h[nbuf] = pltpu.async_copy(
            idx_hbm.at[pl.ds(base_of(step + 1), BLK)],
            idx_bufs[nbuf], isems[nbuf])
      idx_h[buf].wait()
      if store_h[buf] is not None:
        store_h[buf].wait()
      ghs = []
      off = 0
      for g in GCHUNKS:
        ghs.append(pltpu.async_copy(
            table_v.at[idx_bufs[buf].at[pl.ds(off, g)]],
            rows_v.at[buf, pl.ds(off, g)],
            gsems[buf]))
        off += g
      for h in ghs:
        h.wait()
      store_h[buf] = pltpu.async_copy(
          rows_v.at[buf], out_hbm.at[pl.ds(base, BLK)], ssems[buf])

    store_h[0].wait()
    store_h[1].wait()

  return k(idx, table)


def kernel(node_degree, degree_emb):
  return _sc_lookup(node_degree.astype(jnp.int32), degree_emb)
